# Initial kernel scaffold; baseline (speedup 1.0000x reference)
import jax
import jax.numpy as jnp
from jax.experimental import pallas as pl


def kernel(x, edge_index, W1l, W1r, b1, W2l, W2r, b2):
    raise NotImplementedError("write your pallas kernel here")


# speedup vs baseline: 7.3391x; 7.3391x over previous
import jax
import jax.numpy as jnp
from jax import lax
from jax.experimental import pallas as pl
from jax.experimental.pallas import tpu as pltpu
from jax.experimental.pallas import tpu_sc as plsc

N = 10000
E = 320000
F_IN = 128
H = 128
C_OUT = 47
D2 = 48

NC = 2
NS = 16
NW = NC * NS
LANES = 16

CHUNK = 128
NCHUNKS = E // CHUNK
BASE_CH = NCHUNKS // NW
EXTRA_CH = NCHUNKS - BASE_CH * NW
N_PAD = 10240
ROWS_PER_TILE = N_PAD // NS
ZROWS = 64


def _make_sc_agg(d, with_deg):
    mesh = plsc.VectorSubcoreMesh(core_axis_name="c", subcore_axis_name="s")
    out_type = [jax.ShapeDtypeStruct((NC, N_PAD, d), jnp.float32)]
    scratch = [
        pltpu.VMEM_SHARED((N_PAD, d), jnp.float32),
        pltpu.VMEM((2, CHUNK), jnp.int32),
        pltpu.VMEM((CHUNK, d), jnp.float32),
        pltpu.VMEM((ZROWS, d), jnp.float32),
        pltpu.SemaphoreType.DMA,
    ]
    if with_deg:
        out_type.append(jax.ShapeDtypeStruct((NC, N_PAD, LANES), jnp.float32))
        scratch += [
            pltpu.VMEM_SHARED((N_PAD, LANES), jnp.float32),
            pltpu.VMEM((CHUNK, LANES), jnp.float32),
            pltpu.VMEM((ZROWS, LANES), jnp.float32),
        ]

    def body(xw, src, dst, *refs):
        if with_deg:
            (agg_out, deg_out, acc, idx_v, rows_v, zbuf, sem,
             dacc, ones_v, zdeg) = refs
        else:
            agg_out, acc, idx_v, rows_v, zbuf, sem = refs
        c = lax.axis_index("c")
        s = lax.axis_index("s")
        wid = c * NS + s
        _F32_16 = jnp.zeros((LANES,), jnp.float32)

        @pl.loop(0, ZROWS)
        def _(i):
            @pl.loop(0, d // LANES)
            def _(j):
                zbuf[i, pl.ds(j * LANES, LANES)] = _F32_16

        if with_deg:
            @pl.loop(0, ZROWS)
            def _(i):
                zdeg[i, pl.ds(0, LANES)] = _F32_16

            @pl.loop(0, CHUNK)
            def _(i):
                ones_v[i, pl.ds(0, LANES)] = _F32_16 + 1.0

        r0 = s * ROWS_PER_TILE

        @pl.loop(0, ROWS_PER_TILE // ZROWS)
        def _(k):
            pltpu.sync_copy(zbuf, acc.at[pl.ds(r0 + k * ZROWS, ZROWS)])

        if with_deg:
            @pl.loop(0, ROWS_PER_TILE // ZROWS)
            def _(k):
                pltpu.sync_copy(zdeg, dacc.at[pl.ds(r0 + k * ZROWS, ZROWS)])

        plsc.subcore_barrier()

        nch = BASE_CH + jnp.where(wid < EXTRA_CH, 1, 0)

        @pl.loop(0, nch)
        def _(k):
            base = (wid + NW * k) * CHUNK
            pltpu.sync_copy(src.at[pl.ds(base, CHUNK)], idx_v.at[0])
            pltpu.sync_copy(dst.at[pl.ds(base, CHUNK)], idx_v.at[1])
            pltpu.async_copy(xw.at[idx_v.at[0]], rows_v, sem).wait()
            pltpu.sync_copy(rows_v, acc.at[idx_v.at[1]], add=True)
            if with_deg:
                pltpu.sync_copy(ones_v, dacc.at[idx_v.at[1]], add=True)

        plsc.subcore_barrier()

        pltpu.sync_copy(acc.at[pl.ds(r0, ROWS_PER_TILE)],
                        agg_out.at[c, pl.ds(r0, ROWS_PER_TILE)])
        if with_deg:
            pltpu.sync_copy(dacc.at[pl.ds(r0, ROWS_PER_TILE)],
                            deg_out.at[c, pl.ds(r0, ROWS_PER_TILE)])

    return pl.kernel(
        body, out_type=out_type, mesh=mesh, scratch_types=scratch,
        compiler_params=pltpu.CompilerParams(use_tc_tiling_on_sc=False))


_sc_agg_deg = _make_sc_agg(H, True)
_sc_agg2 = _make_sc_agg(D2, False)

TC_BLK = 2000


def _lin2_body(x_ref, wl_ref, wr_ref, b_ref, xl_ref, xr_ref):
    x = x_ref[...]
    xl_ref[...] = jnp.dot(x, wl_ref[...], preferred_element_type=jnp.float32)
    xr_ref[...] = (jnp.dot(x, wr_ref[...], preferred_element_type=jnp.float32)
                   + b_ref[...])


def _lin2(x, wl, wr, b):
    return pl.pallas_call(
        _lin2_body,
        grid=(N // TC_BLK,),
        in_specs=[
            pl.BlockSpec((TC_BLK, F_IN), lambda i: (i, 0)),
            pl.BlockSpec((F_IN, H), lambda i: (0, 0)),
            pl.BlockSpec((F_IN, H), lambda i: (0, 0)),
            pl.BlockSpec((1, H), lambda i: (0, 0)),
        ],
        out_specs=[
            pl.BlockSpec((TC_BLK, H), lambda i: (i, 0)),
            pl.BlockSpec((TC_BLK, H), lambda i: (i, 0)),
        ],
        out_shape=[
            jax.ShapeDtypeStruct((N, H), jnp.float32),
            jax.ShapeDtypeStruct((N, H), jnp.float32),
        ],
    )(x, wl, wr, b)


def _mid_body(aggp_ref, degp_ref, xr_ref, w2l_ref, w2r_ref, b2_ref,
              hl_ref, hr_ref):
    agg = aggp_ref[0] + aggp_ref[1]
    deg = degp_ref[0, :, 0:1] + degp_ref[1, :, 0:1]
    h = jnp.maximum(agg / jnp.maximum(deg, 1.0) + xr_ref[...], 0.0)
    hl_ref[...] = jnp.dot(h, w2l_ref[...], preferred_element_type=jnp.float32)
    hr_ref[...] = (jnp.dot(h, w2r_ref[...], preferred_element_type=jnp.float32)
                   + b2_ref[...])


def _mid(aggp, degp, xr, w2l, w2r, b2):
    return pl.pallas_call(
        _mid_body,
        grid=(N // TC_BLK,),
        in_specs=[
            pl.BlockSpec((NC, TC_BLK, H), lambda i: (0, i, 0)),
            pl.BlockSpec((NC, TC_BLK, LANES), lambda i: (0, i, 0)),
            pl.BlockSpec((TC_BLK, H), lambda i: (i, 0)),
            pl.BlockSpec((H, D2), lambda i: (0, 0)),
            pl.BlockSpec((H, D2), lambda i: (0, 0)),
            pl.BlockSpec((1, D2), lambda i: (0, 0)),
        ],
        out_specs=[
            pl.BlockSpec((TC_BLK, D2), lambda i: (i, 0)),
            pl.BlockSpec((TC_BLK, D2), lambda i: (i, 0)),
        ],
        out_shape=[
            jax.ShapeDtypeStruct((N, D2), jnp.float32),
            jax.ShapeDtypeStruct((N, D2), jnp.float32),
        ],
    )(aggp, degp, xr, w2l, w2r, b2)


def _out_body(aggp_ref, degp_ref, hr_ref, o_ref):
    agg = aggp_ref[0] + aggp_ref[1]
    deg = degp_ref[0, :, 0:1] + degp_ref[1, :, 0:1]
    o_ref[...] = agg / jnp.maximum(deg, 1.0) + hr_ref[...]


def _outk(aggp, degp, hr):
    return pl.pallas_call(
        _out_body,
        grid=(N // TC_BLK,),
        in_specs=[
            pl.BlockSpec((NC, TC_BLK, D2), lambda i: (0, i, 0)),
            pl.BlockSpec((NC, TC_BLK, LANES), lambda i: (0, i, 0)),
            pl.BlockSpec((TC_BLK, D2), lambda i: (i, 0)),
        ],
        out_specs=pl.BlockSpec((TC_BLK, D2), lambda i: (i, 0)),
        out_shape=jax.ShapeDtypeStruct((N, D2), jnp.float32),
    )(aggp, degp, hr)


@jax.jit
def kernel(x, edge_index, W1l, W1r, b1, W2l, W2r, b2):
    src = edge_index[0]
    dst = edge_index[1]
    xl, xr = _lin2(x, W1l, W1r, b1.reshape(1, H))
    aggp, degp = _sc_agg_deg(xl, src, dst)
    w2l_p = jnp.pad(W2l, ((0, 0), (0, D2 - C_OUT)))
    w2r_p = jnp.pad(W2r, ((0, 0), (0, D2 - C_OUT)))
    b2_p = jnp.pad(b2, (0, D2 - C_OUT)).reshape(1, D2)
    hl, hr = _mid(aggp, degp, xr, w2l_p, w2r_p, b2_p)
    agg2p, = _sc_agg2(hl, src, dst)
    out48 = _outk(agg2p, degp, hr)
    return out48[:, :C_OUT]


# speedup vs baseline: 10.5622x; 1.4392x over previous
import jax
import jax.numpy as jnp
from jax import lax
from jax.experimental import pallas as pl
from jax.experimental.pallas import tpu as pltpu
from jax.experimental.pallas import tpu_sc as plsc

N = 10000
E = 320000
F_IN = 128
H = 128
C_OUT = 47
D1 = 144
D2 = 48

NC = 2
NS = 16
NW = NC * NS
LANES = 16

N_PAD = 10240
ROWS_PER_TILE = N_PAD // NS

NBUF = 3
NIDX = 6


def _make_sc_agg(d, chunk):
    nchunks = E // chunk
    base_ch = nchunks // NW
    extra_ch = nchunks - base_ch * NW
    max_ch = base_ch + (1 if extra_ch else 0)
    zrows = min(chunk, ROWS_PER_TILE)
    nzcopy = ROWS_PER_TILE // zrows
    assert ROWS_PER_TILE % zrows == 0 and chunk % 8 == 0 and d % LANES == 0

    mesh = plsc.VectorSubcoreMesh(core_axis_name="c", subcore_axis_name="s")
    out_type = [jax.ShapeDtypeStruct((NC, N_PAD, d), jnp.float32)]
    scratch = (
        [pltpu.VMEM_SHARED((N_PAD, d), jnp.float32)]
        + [pltpu.VMEM((chunk, d), jnp.float32) for _ in range(NBUF)]
        + [pltpu.VMEM((2, chunk), jnp.int32) for _ in range(NIDX)]
        + [pltpu.SemaphoreType.DMA for _ in range(2 * NBUF + NIDX)]
    )

    def body(xw, src, dst, agg_out, acc, *refs):
        rows = refs[:NBUF]
        idx = refs[NBUF:NBUF + NIDX]
        sem_g = refs[NBUF + NIDX:2 * NBUF + NIDX]
        sem_s = refs[2 * NBUF + NIDX:3 * NBUF + NIDX]
        sem_i = refs[3 * NBUF + NIDX:]
        c = lax.axis_index("c")
        s = lax.axis_index("s")
        wid = c * NS + s
        nch = base_ch + jnp.where(wid < extra_ch, 1, 0) if extra_ch else base_ch

        zero = jnp.zeros((LANES,), jnp.float32)

        @pl.loop(0, zrows)
        def _(i):
            @pl.loop(0, d // LANES)
            def _(j):
                rows[0][i, pl.ds(j * LANES, LANES)] = zero

        r0 = s * ROWS_PER_TILE

        @pl.loop(0, nzcopy)
        def _(k):
            pltpu.sync_copy(rows[0].at[pl.ds(0, zrows)],
                            acc.at[pl.ds(r0 + k * zrows, zrows)])

        plsc.subcore_barrier()

        def issue_idx(k, ib):
            base = (wid + NW * k) * chunk
            pltpu.async_copy(src.at[pl.ds(base, chunk)], idx[ib].at[0],
                             sem_i[ib])
            pltpu.async_copy(dst.at[pl.ds(base, chunk)], idx[ib].at[1],
                             sem_i[ib])

        def wait_idx(k, ib):
            base = (wid + NW * k) * chunk
            pltpu.make_async_copy(src.at[pl.ds(base, chunk)], idx[ib].at[0],
                                  sem_i[ib]).wait()
            pltpu.make_async_copy(dst.at[pl.ds(base, chunk)], idx[ib].at[1],
                                  sem_i[ib]).wait()

        def issue_gather(b, ib):
            pltpu.async_copy(xw.at[idx[ib].at[0]], rows[b], sem_g[b])

        def wait_gather(b, ib):
            pltpu.make_async_copy(xw.at[idx[ib].at[0]], rows[b],
                                  sem_g[b]).wait()

        def issue_scatter(b, ib):
            pltpu.async_copy(rows[b], acc.at[idx[ib].at[1]], sem_s[b],
                             add=True)

        def wait_scatter(b, ib):
            pltpu.make_async_copy(rows[b], acc.at[idx[ib].at[1]],
                                  sem_s[b]).wait()

        issue_idx(0, 0)
        issue_idx(1, 1)
        wait_idx(0, 0)
        issue_gather(0, 0)

        nslots = max_ch + NBUF
        nouter = (nslots + NIDX - 1) // NIDX

        @pl.loop(0, nouter)
        def _(io):
            k0 = io * NIDX
            for j in range(NIDX):
                k = k0 + j
                b = j % NBUF
                bn = (j + 1) % NBUF
                ibn = (j + 1) % NIDX
                ib2 = (j + 2) % NIDX

                @pl.when(k < nch)
                def _():
                    wait_gather(b, j)
                    issue_scatter(b, j)

                @pl.when(k + 1 < nch)
                def _():
                    wait_idx(k + 1, ibn)

                @pl.when((k >= 2) & (k - 2 < nch))
                def _():
                    wait_scatter((j + 1) % NBUF, (j - 2) % NIDX)

                @pl.when(k + 1 < nch)
                def _():
                    issue_gather(bn, ibn)

                @pl.when(k + 2 < nch)
                def _():
                    issue_idx(k + 2, ib2)

        plsc.subcore_barrier()

        pltpu.sync_copy(acc.at[pl.ds(r0, ROWS_PER_TILE)],
                        agg_out.at[c, pl.ds(r0, ROWS_PER_TILE)])

    return pl.kernel(
        body, out_type=out_type, mesh=mesh, scratch_types=scratch,
        compiler_params=pltpu.CompilerParams(use_tc_tiling_on_sc=False))


_sc_agg1 = _make_sc_agg(D1, 80)
_sc_agg2 = _make_sc_agg(D2, 128)

TC_BLK = 2000


def _lin2_body(x_ref, wl_ref, wr_ref, b_ref, xl_ref, xr_ref):
    x = x_ref[...]
    xl_ref[:, :H] = jnp.dot(x, wl_ref[...], preferred_element_type=jnp.float32)
    xl_ref[:, H:] = jnp.ones((TC_BLK, D1 - H), jnp.float32)
    xr_ref[...] = (jnp.dot(x, wr_ref[...], preferred_element_type=jnp.float32)
                   + b_ref[...])


def _lin2(x, wl, wr, b):
    return pl.pallas_call(
        _lin2_body,
        grid=(N // TC_BLK,),
        in_specs=[
            pl.BlockSpec((TC_BLK, F_IN), lambda i: (i, 0)),
            pl.BlockSpec((F_IN, H), lambda i: (0, 0)),
            pl.BlockSpec((F_IN, H), lambda i: (0, 0)),
            pl.BlockSpec((1, H), lambda i: (0, 0)),
        ],
        out_specs=[
            pl.BlockSpec((TC_BLK, D1), lambda i: (i, 0)),
            pl.BlockSpec((TC_BLK, H), lambda i: (i, 0)),
        ],
        out_shape=[
            jax.ShapeDtypeStruct((N, D1), jnp.float32),
            jax.ShapeDtypeStruct((N, H), jnp.float32),
        ],
    )(x, wl, wr, b)


def _mid_body(aggp_ref, xr_ref, w2l_ref, w2r_ref, b2_ref,
              hl_ref, hr_ref):
    agg = aggp_ref[0, :, :H] + aggp_ref[1, :, :H]
    deg = aggp_ref[0, :, H:H + 1] + aggp_ref[1, :, H:H + 1]
    h = jnp.maximum(agg / jnp.maximum(deg, 1.0) + xr_ref[...], 0.0)
    hl_ref[...] = jnp.dot(h, w2l_ref[...], preferred_element_type=jnp.float32)
    hr_ref[...] = (jnp.dot(h, w2r_ref[...], preferred_element_type=jnp.float32)
                   + b2_ref[...])


def _mid(aggp, xr, w2l, w2r, b2):
    return pl.pallas_call(
        _mid_body,
        grid=(N // TC_BLK,),
        in_specs=[
            pl.BlockSpec((NC, TC_BLK, D1), lambda i: (0, i, 0)),
            pl.BlockSpec((TC_BLK, H), lambda i: (i, 0)),
            pl.BlockSpec((H, D2), lambda i: (0, 0)),
            pl.BlockSpec((H, D2), lambda i: (0, 0)),
            pl.BlockSpec((1, D2), lambda i: (0, 0)),
        ],
        out_specs=[
            pl.BlockSpec((TC_BLK, D2), lambda i: (i, 0)),
            pl.BlockSpec((TC_BLK, D2), lambda i: (i, 0)),
        ],
        out_shape=[
            jax.ShapeDtypeStruct((N, D2), jnp.float32),
            jax.ShapeDtypeStruct((N, D2), jnp.float32),
        ],
    )(aggp, xr, w2l, w2r, b2)


def _out_body(aggp_ref, degp_ref, hr_ref, o_ref):
    agg = aggp_ref[0] + aggp_ref[1]
    deg = degp_ref[0, :, H:H + 1] + degp_ref[1, :, H:H + 1]
    o_ref[...] = agg / jnp.maximum(deg, 1.0) + hr_ref[...]


def _outk(agg2p, agg1p, hr):
    return pl.pallas_call(
        _out_body,
        grid=(N // TC_BLK,),
        in_specs=[
            pl.BlockSpec((NC, TC_BLK, D2), lambda i: (0, i, 0)),
            pl.BlockSpec((NC, TC_BLK, D1), lambda i: (0, i, 0)),
            pl.BlockSpec((TC_BLK, D2), lambda i: (i, 0)),
        ],
        out_specs=pl.BlockSpec((TC_BLK, D2), lambda i: (i, 0)),
        out_shape=jax.ShapeDtypeStruct((N, D2), jnp.float32),
    )(agg2p, agg1p, hr)


@jax.jit
def kernel(x, edge_index, W1l, W1r, b1, W2l, W2r, b2):
    src = edge_index[0]
    dst = edge_index[1]
    xl, xr = _lin2(x, W1l, W1r, b1.reshape(1, H))
    agg1p, = _sc_agg1(xl, src, dst)
    w2l_p = jnp.pad(W2l, ((0, 0), (0, D2 - C_OUT)))
    w2r_p = jnp.pad(W2r, ((0, 0), (0, D2 - C_OUT)))
    b2_p = jnp.pad(b2, (0, D2 - C_OUT)).reshape(1, D2)
    hl, hr = _mid(agg1p, xr, w2l_p, w2r_p, b2_p)
    agg2p, = _sc_agg2(hl, src, dst)
    out48 = _outk(agg2p, agg1p, hr)
    return out48[:, :C_OUT]


# speedup vs baseline: 11.7243x; 1.1100x over previous
import jax
import jax.numpy as jnp
from jax import lax
from jax.experimental import pallas as pl
from jax.experimental.pallas import tpu as pltpu
from jax.experimental.pallas import tpu_sc as plsc

N = 10000
E = 320000
F_IN = 128
H = 128
C_OUT = 47
D1 = 144
D2 = 48

NC = 2
NS = 16
NW = NC * NS
LANES = 16

N_PAD = 10240
ROWS_PER_TILE = N_PAD // NS


def _make_sc_agg(d, chunk, nbuf, nidx):
    nchunks = E // chunk
    base_ch = nchunks // NW
    extra_ch = nchunks - base_ch * NW
    max_ch = base_ch + (1 if extra_ch else 0)
    zrows = min(chunk, ROWS_PER_TILE)
    nzcopy = ROWS_PER_TILE // zrows
    assert ROWS_PER_TILE % zrows == 0 and chunk % 8 == 0 and d % LANES == 0
    assert nidx % nbuf == 0 and nidx >= nbuf + 1 and E == nchunks * chunk

    mesh = plsc.VectorSubcoreMesh(core_axis_name="c", subcore_axis_name="s")
    out_type = [jax.ShapeDtypeStruct((NC, N_PAD, d), jnp.float32)]
    scratch = (
        [pltpu.VMEM_SHARED((N_PAD, d), jnp.float32)]
        + [pltpu.VMEM((chunk, d), jnp.float32) for _ in range(nbuf)]
        + [pltpu.VMEM((2, chunk), jnp.int32) for _ in range(nidx)]
        + [pltpu.SemaphoreType.DMA for _ in range(2 * nbuf + nidx)]
    )

    def body(xw, ei, agg_out, acc, *refs):
        rows = refs[:nbuf]
        idx = refs[nbuf:nbuf + nidx]
        sem_g = refs[nbuf + nidx:2 * nbuf + nidx]
        sem_s = refs[2 * nbuf + nidx:3 * nbuf + nidx]
        sem_i = refs[3 * nbuf + nidx:]
        c = lax.axis_index("c")
        s = lax.axis_index("s")
        wid = c * NS + s
        nch = base_ch + jnp.where(wid < extra_ch, 1, 0) if extra_ch else base_ch

        zero = jnp.zeros((LANES,), jnp.float32)

        @pl.loop(0, zrows)
        def _(i):
            @pl.loop(0, d // LANES)
            def _(j):
                rows[0][i, pl.ds(j * LANES, LANES)] = zero

        r0 = s * ROWS_PER_TILE

        @pl.loop(0, nzcopy)
        def _(k):
            pltpu.sync_copy(rows[0].at[pl.ds(0, zrows)],
                            acc.at[pl.ds(r0 + k * zrows, zrows)])

        plsc.subcore_barrier()

        def issue_idx(k, ib):
            base = (wid + NW * k) * chunk
            pltpu.async_copy(ei.at[0, pl.ds(base, chunk)], idx[ib].at[0],
                             sem_i[ib])
            pltpu.async_copy(ei.at[1, pl.ds(base, chunk)], idx[ib].at[1],
                             sem_i[ib])

        def wait_idx(k, ib):
            base = (wid + NW * k) * chunk
            pltpu.make_async_copy(ei.at[0, pl.ds(base, chunk)], idx[ib].at[0],
                                  sem_i[ib]).wait()
            pltpu.make_async_copy(ei.at[1, pl.ds(base, chunk)], idx[ib].at[1],
                                  sem_i[ib]).wait()

        def issue_gather(b, ib):
            pltpu.async_copy(xw.at[idx[ib].at[0]], rows[b], sem_g[b])

        def wait_gather(b, ib):
            pltpu.make_async_copy(xw.at[idx[ib].at[0]], rows[b],
                                  sem_g[b]).wait()

        def issue_scatter(b, ib):
            pltpu.async_copy(rows[b], acc.at[idx[ib].at[1]], sem_s[b],
                             add=True)

        def wait_scatter(b, ib):
            pltpu.make_async_copy(rows[b], acc.at[idx[ib].at[1]],
                                  sem_s[b]).wait()

        issue_idx(0, 0)
        issue_idx(1, 1)
        wait_idx(0, 0)
        issue_gather(0, 0)

        nslots = max_ch + nbuf
        nouter = (nslots + nidx - 1) // nidx

        @pl.loop(0, nouter)
        def _(io):
            k0 = io * nidx
            for j in range(nidx):
                k = k0 + j
                b = j % nbuf
                bn = (j + 1) % nbuf
                ibn = (j + 1) % nidx
                ib2 = (j + 2) % nidx

                @pl.when(k < nch)
                def _():
                    wait_gather(b, j)
                    issue_scatter(b, j)

                @pl.when(k + 1 < nch)
                def _():
                    wait_idx(k + 1, ibn)

                @pl.when((k >= nbuf - 1) & (k + 1 - nbuf < nch))
                def _():
                    wait_scatter(bn, (j + 1 - nbuf) % nidx)

                @pl.when(k + 1 < nch)
                def _():
                    issue_gather(bn, ibn)

                @pl.when(k + 2 < nch)
                def _():
                    issue_idx(k + 2, ib2)

        plsc.subcore_barrier()

        pltpu.sync_copy(acc.at[pl.ds(r0, ROWS_PER_TILE)],
                        agg_out.at[c, pl.ds(r0, ROWS_PER_TILE)])

    return pl.kernel(
        body, out_type=out_type, mesh=mesh, scratch_types=scratch,
        compiler_params=pltpu.CompilerParams(use_tc_tiling_on_sc=False))


_sc_agg1 = _make_sc_agg(D1, 128, 2, 6)
_sc_agg2 = _make_sc_agg(D2, 128, 4, 8)

TC_BLK = 2000


def _lin2_body(x_ref, wl_ref, wr_ref, b_ref, xl_ref, xr_ref):
    x = x_ref[...]
    xl_ref[:, :H] = jnp.dot(x, wl_ref[...], preferred_element_type=jnp.float32)
    xl_ref[:, H:] = jnp.ones((TC_BLK, D1 - H), jnp.float32)
    xr_ref[...] = (jnp.dot(x, wr_ref[...], preferred_element_type=jnp.float32)
                   + b_ref[...])


def _lin2(x, wl, wr, b):
    return pl.pallas_call(
        _lin2_body,
        grid=(N // TC_BLK,),
        in_specs=[
            pl.BlockSpec((TC_BLK, F_IN), lambda i: (i, 0)),
            pl.BlockSpec((F_IN, H), lambda i: (0, 0)),
            pl.BlockSpec((F_IN, H), lambda i: (0, 0)),
            pl.BlockSpec((1, H), lambda i: (0, 0)),
        ],
        out_specs=[
            pl.BlockSpec((TC_BLK, D1), lambda i: (i, 0)),
            pl.BlockSpec((TC_BLK, H), lambda i: (i, 0)),
        ],
        out_shape=[
            jax.ShapeDtypeStruct((N, D1), jnp.float32),
            jax.ShapeDtypeStruct((N, H), jnp.float32),
        ],
    )(x, wl, wr, b)


def _mid_body(aggp_ref, xr_ref, w2l_ref, w2r_ref, b2_ref, hl_ref, hr_ref):
    agg = aggp_ref[0, :, :H] + aggp_ref[1, :, :H]
    deg = aggp_ref[0, :, H:H + 1] + aggp_ref[1, :, H:H + 1]
    h = jnp.maximum(agg / jnp.maximum(deg, 1.0) + xr_ref[...], 0.0)
    zcol = jnp.zeros((H, 1), jnp.float32)
    w2l = jnp.concatenate([w2l_ref[...], zcol], axis=1)
    w2r = jnp.concatenate([w2r_ref[...], zcol], axis=1)
    b2 = jnp.concatenate([b2_ref[...], jnp.zeros((1, 1), jnp.float32)], axis=1)
    hl_ref[...] = jnp.dot(h, w2l, preferred_element_type=jnp.float32)
    hr_ref[...] = jnp.dot(h, w2r, preferred_element_type=jnp.float32) + b2


def _mid(aggp, xr, w2l, w2r, b2):
    return pl.pallas_call(
        _mid_body,
        grid=(N // TC_BLK,),
        in_specs=[
            pl.BlockSpec((NC, TC_BLK, D1), lambda i: (0, i, 0)),
            pl.BlockSpec((TC_BLK, H), lambda i: (i, 0)),
            pl.BlockSpec((H, C_OUT), lambda i: (0, 0)),
            pl.BlockSpec((H, C_OUT), lambda i: (0, 0)),
            pl.BlockSpec((1, C_OUT), lambda i: (0, 0)),
        ],
        out_specs=[
            pl.BlockSpec((TC_BLK, D2), lambda i: (i, 0)),
            pl.BlockSpec((TC_BLK, D2), lambda i: (i, 0)),
        ],
        out_shape=[
            jax.ShapeDtypeStruct((N, D2), jnp.float32),
            jax.ShapeDtypeStruct((N, D2), jnp.float32),
        ],
    )(aggp, xr, w2l, w2r, b2)


def _out_body(aggp_ref, degp_ref, hr_ref, o_ref):
    agg = aggp_ref[0] + aggp_ref[1]
    deg = degp_ref[0, :, H:H + 1] + degp_ref[1, :, H:H + 1]
    res = agg / jnp.maximum(deg, 1.0) + hr_ref[...]
    o_ref[...] = res[:, :C_OUT]


def _outk(agg2p, agg1p, hr):
    return pl.pallas_call(
        _out_body,
        grid=(N // TC_BLK,),
        in_specs=[
            pl.BlockSpec((NC, TC_BLK, D2), lambda i: (0, i, 0)),
            pl.BlockSpec((NC, TC_BLK, D1), lambda i: (0, i, 0)),
            pl.BlockSpec((TC_BLK, D2), lambda i: (i, 0)),
        ],
        out_specs=pl.BlockSpec((TC_BLK, C_OUT), lambda i: (i, 0)),
        out_shape=jax.ShapeDtypeStruct((N, C_OUT), jnp.float32),
    )(agg2p, agg1p, hr)


@jax.jit
def kernel(x, edge_index, W1l, W1r, b1, W2l, W2r, b2):
    xl, xr = _lin2(x, W1l, W1r, b1.reshape(1, H))
    agg1p, = _sc_agg1(xl, edge_index)
    hl, hr = _mid(agg1p, xr, W2l, W2r, b2.reshape(1, C_OUT))
    agg2p, = _sc_agg2(hl, edge_index)
    return _outk(agg2p, agg1p, hr)
